# Initial kernel scaffold; baseline (speedup 1.0000x reference)
import jax
import jax.numpy as jnp
from jax.experimental import pallas as pl


def kernel(x_author, x_paper, edge_index_ap, edge_index_pa, W_author, b_author, W_paper, b_paper, Wq, bq, Wk, bk):
    raise NotImplementedError("write your pallas kernel here")


# speedup vs baseline: 2.3432x; 2.3432x over previous
import functools

import jax
import jax.numpy as jnp
from jax import lax
from jax.experimental import pallas as pl
from jax.experimental.pallas import tpu as pltpu
from jax.experimental.pallas import tpu_sc as plsc

_NC = 2
_NS = 16
_L = 16
_NW = _NC * _NS


def _proj_body(xa, xp, wa, ba, wp, bp, wq, bq, wk, bk, qa, ka, qp, kp):
    ha = jnp.dot(xa[...], wa[...], preferred_element_type=jnp.float32) + ba[...]
    hp = jnp.dot(xp[...], wp[...], preferred_element_type=jnp.float32) + bp[...]
    qa[...] = jnp.dot(ha, wq[...], preferred_element_type=jnp.float32) + bq[...]
    ka[...] = jnp.dot(ha, wk[...], preferred_element_type=jnp.float32) + bk[...]
    qp[...] = jnp.dot(hp, wq[...], preferred_element_type=jnp.float32) + bq[...]
    kp[...] = jnp.dot(hp, wk[...], preferred_element_type=jnp.float32) + bk[...]


def _projections(x_author, x_paper, W_author, b_author, W_paper, b_paper,
                 Wq, bq, Wk, bk):
    N, D = x_author.shape
    H = W_author.shape[1]
    BM = 2000
    grid = (N // BM,)
    x_spec = pl.BlockSpec((BM, D), lambda i: (i, 0))
    w_spec = pl.BlockSpec((D, H), lambda i: (0, 0))
    w2_spec = pl.BlockSpec((H, H), lambda i: (0, 0))
    b_spec = pl.BlockSpec((1, H), lambda i: (0, 0))
    o_spec = pl.BlockSpec((BM, H), lambda i: (i, 0))
    out_sds = jax.ShapeDtypeStruct((N, H), jnp.float32)
    return pl.pallas_call(
        _proj_body,
        grid=grid,
        in_specs=[x_spec, x_spec, w_spec, b_spec, w_spec, b_spec,
                  w2_spec, b_spec, w2_spec, b_spec],
        out_specs=[o_spec, o_spec, o_spec, o_spec],
        out_shape=[out_sds, out_sds, out_sds, out_sds],
    )(x_author, x_paper, W_author, b_author.reshape(1, H),
      W_paper, b_paper.reshape(1, H), Wq, bq.reshape(1, H),
      Wk, bk.reshape(1, H))


def _make_edge_kernel(N, H, E):
    EPW = E // _NW
    C = 80
    NCHUNK = EPW // C
    GROUPS = C // _L
    HV = H // _L
    mesh = plsc.VectorSubcoreMesh(core_axis_name="c", subcore_axis_name="s")

    @functools.partial(
        pl.kernel,
        mesh=mesh,
        compiler_params=pltpu.CompilerParams(needs_layout_passes=False),
        out_type=jax.ShapeDtypeStruct((2, 2, _NW, _L), jnp.float32),
        scratch_types=[
            pltpu.VMEM((C,), jnp.int32),
            pltpu.VMEM((C,), jnp.int32),
            pltpu.VMEM((C, H), jnp.float32),
            pltpu.VMEM((C, H), jnp.float32),
            pltpu.VMEM((_L,), jnp.float32),
            pltpu.VMEM((_L,), jnp.float32),
            pltpu.SemaphoreType.DMA,
            pltpu.SemaphoreType.DMA,
        ],
    )
    def edge_kernel(qa, kp, qp, ka, sap, dap, spa, dpa, ms_out,
                    sidx_c, didx_c, qrows, krows, mstage, sstage,
                    semq, semk):
        wid = lax.axis_index("s") * _NC + lax.axis_index("c")
        base = wid * EPW
        inv_sqrt_h = 1.0 / float(H) ** 0.5
        iota16 = lax.iota(jnp.int32, _L)
        for r in range(2):
            qt = (qa, qp)[r]
            kt = (kp, ka)[r]
            st = (sap, spa)[r]
            dt = (dap, dpa)[r]
            mstage[...] = jnp.full((_L,), -1e30, jnp.float32)
            sstage[...] = jnp.zeros((_L,), jnp.float32)

            def chunk_body(c, carry):
                off = base + c * C
                pltpu.sync_copy(st.at[pl.ds(off, C)], sidx_c)
                pltpu.sync_copy(dt.at[pl.ds(off, C)], didx_c)
                cq = pltpu.async_copy(qt.at[sidx_c], qrows, semq)
                ck = pltpu.async_copy(kt.at[didx_c], krows, semk)
                cq.wait()
                ck.wait()

                def group_body(g, carry2):
                    e0 = g * _L
                    logits = jnp.zeros((_L,), jnp.float32)
                    for e in range(_L):
                        row = e0 + e
                        acc = qrows[row, pl.ds(0, _L)] * krows[row, pl.ds(0, _L)]
                        for j in range(1, HV):
                            acc = acc + (qrows[row, pl.ds(j * _L, _L)] *
                                         krows[row, pl.ds(j * _L, _L)])
                        logits = jnp.where(iota16 == e,
                                           jnp.sum(acc) * inv_sqrt_h, logits)
                    m_old = mstage[...]
                    m_new = jnp.maximum(m_old, logits)
                    sstage[...] = (sstage[...] * jnp.exp(m_old - m_new) +
                                   jnp.exp(logits - m_new))
                    mstage[...] = m_new
                    return carry2

                return lax.fori_loop(0, GROUPS, group_body, carry)

            lax.fori_loop(0, NCHUNK, chunk_body, 0)
            pltpu.sync_copy(mstage, ms_out.at[0, r, wid])
            pltpu.sync_copy(sstage, ms_out.at[1, r, wid])

    return edge_kernel


def _combine_body(ms, out, *, E):
    msv = ms[...]
    m0 = msv[0:1, :]
    m1 = msv[1:2, :]
    s0 = msv[2:3, :]
    s1 = msv[3:4, :]
    M0 = jnp.max(m0)
    M1 = jnp.max(m1)
    S0 = jnp.sum(s0 * jnp.exp(m0 - M0))
    S1 = jnp.sum(s1 * jnp.exp(m1 - M1))
    wm0 = (S0 / S0) * (1.0 / E)
    wm1 = (S1 / S1) * (1.0 / E)

    def sm2(a, b):
        mx = jnp.maximum(a, b)
        ea = jnp.exp(a - mx)
        eb = jnp.exp(b - mx)
        z = ea + eb
        return ea / z, eb / z

    r0, r1 = sm2(wm0, wm1)
    s3a, s3b = sm2(0.1 * r0, 0.1 * r1)
    s4a, s4b = sm2(0.1 * r0 * r1, 0.1 * r1 * r0)
    s5a, s5b = sm2(0.1 * r0 * r1 * r0, 0.1 * r1 * r0 * r1)
    row = lax.broadcasted_iota(jnp.int32, (8, 128), 0)
    col = lax.broadcasted_iota(jnp.int32, (8, 128), 1)
    vals = jnp.zeros((8, 128), jnp.float32)
    for i, v in enumerate((s3a, s3b, s4a, s4b, s5a, s5b)):
        vals = jnp.where((row == 0) & (col == i), v, vals)
    out[...] = vals


def kernel(x_author, x_paper, edge_index_ap, edge_index_pa, W_author,
           b_author, W_paper, b_paper, Wq, bq, Wk, bk):
    N, D = x_author.shape
    H = W_author.shape[1]
    E = edge_index_ap.shape[1]

    qa, ka, qp, kp = _projections(x_author, x_paper, W_author, b_author,
                                  W_paper, b_paper, Wq, bq, Wk, bk)

    edge_kernel = _make_edge_kernel(N, H, E)
    ms = edge_kernel(qa, kp, qp, ka,
                     edge_index_ap[0], edge_index_ap[1],
                     edge_index_pa[0], edge_index_pa[1])

    res = pl.pallas_call(
        functools.partial(_combine_body, E=E),
        out_shape=jax.ShapeDtypeStruct((8, 128), jnp.float32),
    )(ms.reshape(4, _NW * _L))
    return res[0, :6]


# speedup vs baseline: 11.8137x; 5.0417x over previous
import functools

import jax
import jax.numpy as jnp
from jax import lax
from jax.experimental import pallas as pl
from jax.experimental.pallas import tpu as pltpu
from jax.experimental.pallas import tpu_sc as plsc

_NC = 2
_NS = 16
_L = 16
_NW = _NC * _NS


def _pack_bf16_pairs(v):
    h2 = v.shape[1] // 2
    b = lax.bitcast_convert_type(v.astype(jnp.bfloat16), jnp.uint16)
    lo = b[:, :h2].astype(jnp.uint32)
    hi = b[:, h2:].astype(jnp.uint32)
    return lax.bitcast_convert_type((hi << 16) | lo, jnp.int32)


def _proj_body(xa, xp, wa, ba, wp, bp, wq, bq, wk, bk, qa, ka, qp, kp):
    ha = jnp.dot(xa[...], wa[...], preferred_element_type=jnp.float32) + ba[...]
    hp = jnp.dot(xp[...], wp[...], preferred_element_type=jnp.float32) + bp[...]
    qa[...] = _pack_bf16_pairs(
        jnp.dot(ha, wq[...], preferred_element_type=jnp.float32) + bq[...])
    ka[...] = _pack_bf16_pairs(
        jnp.dot(ha, wk[...], preferred_element_type=jnp.float32) + bk[...])
    qp[...] = _pack_bf16_pairs(
        jnp.dot(hp, wq[...], preferred_element_type=jnp.float32) + bq[...])
    kp[...] = _pack_bf16_pairs(
        jnp.dot(hp, wk[...], preferred_element_type=jnp.float32) + bk[...])


def _projections(x_author, x_paper, W_author, b_author, W_paper, b_paper,
                 Wq, bq, Wk, bk):
    N, D = x_author.shape
    H = W_author.shape[1]
    BM = 2000
    grid = (N // BM,)
    x_spec = pl.BlockSpec((BM, D), lambda i: (i, 0))
    w_spec = pl.BlockSpec((D, H), lambda i: (0, 0))
    w2_spec = pl.BlockSpec((H, H), lambda i: (0, 0))
    b_spec = pl.BlockSpec((1, H), lambda i: (0, 0))
    o_spec = pl.BlockSpec((BM, H // 2), lambda i: (i, 0))
    out_sds = jax.ShapeDtypeStruct((N, H // 2), jnp.int32)
    return pl.pallas_call(
        _proj_body,
        grid=grid,
        in_specs=[x_spec, x_spec, w_spec, b_spec, w_spec, b_spec,
                  w2_spec, b_spec, w2_spec, b_spec],
        out_specs=[o_spec, o_spec, o_spec, o_spec],
        out_shape=[out_sds, out_sds, out_sds, out_sds],
    )(x_author, x_paper, W_author, b_author.reshape(1, H),
      W_paper, b_paper.reshape(1, H), Wq, bq.reshape(1, H),
      Wk, bk.reshape(1, H))


def _make_edge_kernel(N, H, E):
    EPW = E // _NW
    C = 80
    NCHUNK = EPW // C
    GROUPS = C // _L
    HW = H // 2
    HV = HW // _L
    mesh = plsc.VectorSubcoreMesh(core_axis_name="c", subcore_axis_name="s")

    @functools.partial(
        pl.kernel,
        mesh=mesh,
        compiler_params=pltpu.CompilerParams(needs_layout_passes=False),
        out_type=jax.ShapeDtypeStruct((2, 2, _NW, _L), jnp.float32),
        scratch_types=[
            pltpu.VMEM((EPW,), jnp.int32),
            pltpu.VMEM((EPW,), jnp.int32),
            pltpu.VMEM((C, HW), jnp.int32),
            pltpu.VMEM((C, HW), jnp.int32),
            pltpu.VMEM((C, HW), jnp.int32),
            pltpu.VMEM((C, HW), jnp.int32),
            pltpu.VMEM((C,), jnp.float32),
            pltpu.VMEM((_L,), jnp.float32),
            pltpu.VMEM((_L,), jnp.float32),
            pltpu.SemaphoreType.DMA,
            pltpu.SemaphoreType.DMA,
            pltpu.SemaphoreType.DMA,
            pltpu.SemaphoreType.DMA,
        ],
    )
    def edge_kernel(qa, kp, qp, ka, sap, dap, spa, dpa, ms_out,
                    sidx, didx, qr0, kr0, qr1, kr1, lbuf, mstage, sstage,
                    semq0, semk0, semq1, semk1):
        wid = lax.axis_index("s") * _NC + lax.axis_index("c")
        base = wid * EPW
        inv_sqrt_h = 1.0 / float(H) ** 0.5
        iota16 = lax.iota(jnp.int32, _L)
        bfly = [jnp.bitwise_xor(iota16, jnp.int32(1 << s)) for s in range(4)]
        mask0 = iota16 == 0
        qrows = (qr0, qr1)
        krows = (kr0, kr1)
        semq = (semq0, semq1)
        semk = (semk0, semk1)
        for r in range(2):
            qt = (qa, qp)[r]
            kt = (kp, ka)[r]
            st = (sap, spa)[r]
            dt = (dap, dpa)[r]
            pltpu.sync_copy(st.at[pl.ds(base, EPW)], sidx)
            pltpu.sync_copy(dt.at[pl.ds(base, EPW)], didx)
            mstage[...] = jnp.full((_L,), -1e30, jnp.float32)
            sstage[...] = jnp.zeros((_L,), jnp.float32)

            def gather(c, buf):
                off = c * C
                pltpu.async_copy(qt.at[sidx.at[pl.ds(off, C)]],
                                 qrows[buf], semq[buf])
                pltpu.async_copy(kt.at[didx.at[pl.ds(off, C)]],
                                 krows[buf], semk[buf])

            def wait(buf):
                pltpu.make_async_copy(qt.at[sidx.at[pl.ds(0, C)]],
                                      qrows[buf], semq[buf]).wait()
                pltpu.make_async_copy(kt.at[didx.at[pl.ds(0, C)]],
                                      krows[buf], semk[buf]).wait()

            def compute(buf):
                qb = qrows[buf]
                kb = krows[buf]

                @plsc.parallel_loop(0, C, step=1, unroll=4)
                def edge_dot(e):
                    acc = (plsc.bitcast(qb[e, pl.ds(0, _L)], jnp.bfloat16) *
                           plsc.bitcast(kb[e, pl.ds(0, _L)], jnp.bfloat16))
                    for j in range(1, HV):
                        acc = acc + (
                            plsc.bitcast(qb[e, pl.ds(j * _L, _L)],
                                         jnp.bfloat16) *
                            plsc.bitcast(kb[e, pl.ds(j * _L, _L)],
                                         jnp.bfloat16))
                    ai = plsc.bitcast(acc, jnp.int32)
                    hi = plsc.bitcast(
                        jnp.bitwise_and(ai, jnp.int32(-65536)), jnp.float32)
                    lo = plsc.bitcast(
                        lax.shift_left(ai, jnp.int32(16)), jnp.float32)
                    f = hi + lo
                    for s in range(4):
                        g = lax.gather(
                            f, bfly[s][:, None],
                            lax.GatherDimensionNumbers(
                                offset_dims=(), collapsed_slice_dims=(0,),
                                start_index_map=(0,)),
                            slice_sizes=(1,),
                            mode=lax.GatherScatterMode.PROMISE_IN_BOUNDS)
                        f = f + g
                    plsc.store_scatter(lbuf, [jnp.full((_L,), e, jnp.int32)],
                                       f, mask=mask0)

                m_vec = mstage[...]
                s_vec = sstage[...]
                for g in range(GROUPS):
                    logits = lbuf[pl.ds(g * _L, _L)] * inv_sqrt_h
                    m_new = jnp.maximum(m_vec, logits)
                    s_vec = (s_vec * jnp.exp(m_vec - m_new) +
                             jnp.exp(logits - m_new))
                    m_vec = m_new
                mstage[...] = m_vec
                sstage[...] = s_vec

            gather(0, 0)

            def pair_body(p, carry):
                c = 2 * p
                gather(c + 1, 1)
                wait(0)
                compute(0)
                gather(c + 2, 0)
                wait(1)
                compute(1)
                return carry

            lax.fori_loop(0, (NCHUNK - 1) // 2, pair_body, 0)
            wait(0)
            compute(0)
            pltpu.sync_copy(mstage, ms_out.at[0, r, wid])
            pltpu.sync_copy(sstage, ms_out.at[1, r, wid])

    return edge_kernel


def _combine_body(ms, out, *, E):
    msv = ms[...]
    m0 = msv[0:1, :]
    m1 = msv[1:2, :]
    s0 = msv[2:3, :]
    s1 = msv[3:4, :]
    M0 = jnp.max(m0)
    M1 = jnp.max(m1)
    S0 = jnp.sum(s0 * jnp.exp(m0 - M0))
    S1 = jnp.sum(s1 * jnp.exp(m1 - M1))
    wm0 = (S0 / S0) * (1.0 / E)
    wm1 = (S1 / S1) * (1.0 / E)

    def sm2(a, b):
        mx = jnp.maximum(a, b)
        ea = jnp.exp(a - mx)
        eb = jnp.exp(b - mx)
        z = ea + eb
        return ea / z, eb / z

    r0, r1 = sm2(wm0, wm1)
    s3a, s3b = sm2(0.1 * r0, 0.1 * r1)
    s4a, s4b = sm2(0.1 * r0 * r1, 0.1 * r1 * r0)
    s5a, s5b = sm2(0.1 * r0 * r1 * r0, 0.1 * r1 * r0 * r1)
    row = lax.broadcasted_iota(jnp.int32, (8, 128), 0)
    col = lax.broadcasted_iota(jnp.int32, (8, 128), 1)
    vals = jnp.zeros((8, 128), jnp.float32)
    for i, v in enumerate((s3a, s3b, s4a, s4b, s5a, s5b)):
        vals = jnp.where((row == 0) & (col == i), v, vals)
    out[...] = vals


def kernel(x_author, x_paper, edge_index_ap, edge_index_pa, W_author,
           b_author, W_paper, b_paper, Wq, bq, Wk, bk):
    N, D = x_author.shape
    H = W_author.shape[1]
    E = edge_index_ap.shape[1]

    qa, ka, qp, kp = _projections(x_author, x_paper, W_author, b_author,
                                  W_paper, b_paper, Wq, bq, Wk, bk)

    edge_kernel = _make_edge_kernel(N, H, E)
    ms = edge_kernel(qa, kp, qp, ka,
                     edge_index_ap[0], edge_index_ap[1],
                     edge_index_pa[0], edge_index_pa[1])

    res = pl.pallas_call(
        functools.partial(_combine_body, E=E),
        out_shape=jax.ShapeDtypeStruct((8, 128), jnp.float32),
    )(ms.reshape(4, _NW * _L))
    return res[0, :6]


# speedup vs baseline: 11.8965x; 1.0070x over previous
import functools

import jax
import jax.numpy as jnp
from jax import lax
from jax.experimental import pallas as pl
from jax.experimental.pallas import tpu as pltpu
from jax.experimental.pallas import tpu_sc as plsc

_NC = 2
_NS = 16
_L = 16
_NW = _NC * _NS


def _pack_bf16_pairs(v):
    h2 = v.shape[1] // 2
    b = lax.bitcast_convert_type(v.astype(jnp.bfloat16), jnp.uint16)
    lo = b[:, :h2].astype(jnp.uint32)
    hi = b[:, h2:].astype(jnp.uint32)
    return lax.bitcast_convert_type((hi << 16) | lo, jnp.int32)


def _proj_body(xa, xp, wa, ba, wp, bp, wq, bq, wk, bk, qa, ka, qp, kp):
    ha = jnp.dot(xa[...], wa[...], preferred_element_type=jnp.float32) + ba[...]
    hp = jnp.dot(xp[...], wp[...], preferred_element_type=jnp.float32) + bp[...]
    qa[...] = _pack_bf16_pairs(
        jnp.dot(ha, wq[...], preferred_element_type=jnp.float32) + bq[...])
    ka[...] = _pack_bf16_pairs(
        jnp.dot(ha, wk[...], preferred_element_type=jnp.float32) + bk[...])
    qp[...] = _pack_bf16_pairs(
        jnp.dot(hp, wq[...], preferred_element_type=jnp.float32) + bq[...])
    kp[...] = _pack_bf16_pairs(
        jnp.dot(hp, wk[...], preferred_element_type=jnp.float32) + bk[...])


def _projections(x_author, x_paper, W_author, b_author, W_paper, b_paper,
                 Wq, bq, Wk, bk):
    N, D = x_author.shape
    H = W_author.shape[1]
    BM = 2000
    grid = (N // BM,)
    x_spec = pl.BlockSpec((BM, D), lambda i: (i, 0))
    w_spec = pl.BlockSpec((D, H), lambda i: (0, 0))
    w2_spec = pl.BlockSpec((H, H), lambda i: (0, 0))
    b_spec = pl.BlockSpec((1, H), lambda i: (0, 0))
    o_spec = pl.BlockSpec((BM, H // 2), lambda i: (i, 0))
    out_sds = jax.ShapeDtypeStruct((N, H // 2), jnp.int32)
    return pl.pallas_call(
        _proj_body,
        grid=grid,
        in_specs=[x_spec, x_spec, w_spec, b_spec, w_spec, b_spec,
                  w2_spec, b_spec, w2_spec, b_spec],
        out_specs=[o_spec, o_spec, o_spec, o_spec],
        out_shape=[out_sds, out_sds, out_sds, out_sds],
    )(x_author, x_paper, W_author, b_author.reshape(1, H),
      W_paper, b_paper.reshape(1, H), Wq, bq.reshape(1, H),
      Wk, bk.reshape(1, H))


def _make_edge_kernel(N, H, E):
    EPW = E // _NW
    C = 80
    NCHUNK = EPW // C
    GROUPS = C // _L
    HW = H // 2
    HV = HW // _L
    mesh = plsc.VectorSubcoreMesh(core_axis_name="c", subcore_axis_name="s")

    @functools.partial(
        pl.kernel,
        mesh=mesh,
        compiler_params=pltpu.CompilerParams(needs_layout_passes=False),
        out_type=jax.ShapeDtypeStruct((2, 2, _NW, _L), jnp.float32),
        scratch_types=[
            pltpu.VMEM((EPW,), jnp.int32),
            pltpu.VMEM((EPW,), jnp.int32),
            pltpu.VMEM((C, HW), jnp.int32),
            pltpu.VMEM((C, HW), jnp.int32),
            pltpu.VMEM((C, HW), jnp.int32),
            pltpu.VMEM((C, HW), jnp.int32),
            pltpu.VMEM((C,), jnp.float32),
            pltpu.VMEM((_L,), jnp.float32),
            pltpu.VMEM((_L,), jnp.float32),
            pltpu.SemaphoreType.DMA,
            pltpu.SemaphoreType.DMA,
            pltpu.SemaphoreType.DMA,
            pltpu.SemaphoreType.DMA,
        ],
    )
    def edge_kernel(qa, kp, qp, ka, sap, dap, spa, dpa, ms_out,
                    sidx, didx, qr0, kr0, qr1, kr1, lbuf, mstage, sstage,
                    semq0, semk0, semq1, semk1):
        wid = lax.axis_index("s") * _NC + lax.axis_index("c")
        base = wid * EPW
        inv_sqrt_h = 1.0 / float(H) ** 0.5
        iota16 = lax.iota(jnp.int32, _L)
        bfly = [jnp.bitwise_xor(iota16, jnp.int32(1 << s)) for s in range(4)]
        mask0 = iota16 == 0
        qrows = (qr0, qr1)
        krows = (kr0, kr1)
        semq = (semq0, semq1)
        semk = (semk0, semk1)
        for r in range(2):
            qt = (qa, qp)[r]
            kt = (kp, ka)[r]
            st = (sap, spa)[r]
            dt = (dap, dpa)[r]
            ci = pltpu.async_copy(st.at[pl.ds(base, EPW)], sidx, semq0)
            pltpu.sync_copy(dt.at[pl.ds(base, EPW)], didx)
            ci.wait()
            mstage[...] = jnp.full((_L,), -1e30, jnp.float32)
            sstage[...] = jnp.zeros((_L,), jnp.float32)

            def gather(c, buf):
                off = c * C
                pltpu.async_copy(qt.at[sidx.at[pl.ds(off, C)]],
                                 qrows[buf], semq[buf])
                pltpu.async_copy(kt.at[didx.at[pl.ds(off, C)]],
                                 krows[buf], semk[buf])

            def wait(buf):
                pltpu.make_async_copy(qt.at[sidx.at[pl.ds(0, C)]],
                                      qrows[buf], semq[buf]).wait()
                pltpu.make_async_copy(kt.at[didx.at[pl.ds(0, C)]],
                                      krows[buf], semk[buf]).wait()

            def compute(buf):
                qb = qrows[buf]
                kb = krows[buf]

                @plsc.parallel_loop(0, C, step=1, unroll=4)
                def edge_dot(e):
                    acc = (plsc.bitcast(qb[e, pl.ds(0, _L)], jnp.bfloat16) *
                           plsc.bitcast(kb[e, pl.ds(0, _L)], jnp.bfloat16))
                    for j in range(1, HV):
                        acc = acc + (
                            plsc.bitcast(qb[e, pl.ds(j * _L, _L)],
                                         jnp.bfloat16) *
                            plsc.bitcast(kb[e, pl.ds(j * _L, _L)],
                                         jnp.bfloat16))
                    ai = plsc.bitcast(acc, jnp.int32)
                    hi = plsc.bitcast(
                        jnp.bitwise_and(ai, jnp.int32(-65536)), jnp.float32)
                    lo = plsc.bitcast(
                        lax.shift_left(ai, jnp.int32(16)), jnp.float32)
                    f = hi + lo
                    for s in range(4):
                        g = lax.gather(
                            f, bfly[s][:, None],
                            lax.GatherDimensionNumbers(
                                offset_dims=(), collapsed_slice_dims=(0,),
                                start_index_map=(0,)),
                            slice_sizes=(1,),
                            mode=lax.GatherScatterMode.PROMISE_IN_BOUNDS)
                        f = f + g
                    plsc.store_scatter(lbuf, [jnp.full((_L,), e, jnp.int32)],
                                       f, mask=mask0)

                m_vec = mstage[...]
                s_vec = sstage[...]
                for g in range(GROUPS):
                    logits = lbuf[pl.ds(g * _L, _L)] * inv_sqrt_h
                    m_new = jnp.maximum(m_vec, logits)
                    s_vec = (s_vec * jnp.exp(m_vec - m_new) +
                             jnp.exp(logits - m_new))
                    m_vec = m_new
                mstage[...] = m_vec
                sstage[...] = s_vec

            gather(0, 0)

            def pair_body(p, carry):
                c = 2 * p
                gather(c + 1, 1)
                wait(0)
                compute(0)
                gather(c + 2, 0)
                wait(1)
                compute(1)
                return carry

            lax.fori_loop(0, (NCHUNK - 1) // 2, pair_body, 0)
            wait(0)
            compute(0)
            pltpu.sync_copy(mstage, ms_out.at[0, r, wid])
            pltpu.sync_copy(sstage, ms_out.at[1, r, wid])

    return edge_kernel


def _combine_body(ms, out, *, E):
    msv = ms[...]
    m0 = msv[0:1, :]
    m1 = msv[1:2, :]
    s0 = msv[2:3, :]
    s1 = msv[3:4, :]
    M0 = jnp.max(m0)
    M1 = jnp.max(m1)
    S0 = jnp.sum(s0 * jnp.exp(m0 - M0))
    S1 = jnp.sum(s1 * jnp.exp(m1 - M1))
    wm0 = (S0 / S0) * (1.0 / E)
    wm1 = (S1 / S1) * (1.0 / E)

    def sm2(a, b):
        mx = jnp.maximum(a, b)
        ea = jnp.exp(a - mx)
        eb = jnp.exp(b - mx)
        z = ea + eb
        return ea / z, eb / z

    r0, r1 = sm2(wm0, wm1)
    s3a, s3b = sm2(0.1 * r0, 0.1 * r1)
    s4a, s4b = sm2(0.1 * r0 * r1, 0.1 * r1 * r0)
    s5a, s5b = sm2(0.1 * r0 * r1 * r0, 0.1 * r1 * r0 * r1)
    row = lax.broadcasted_iota(jnp.int32, (8, 128), 0)
    col = lax.broadcasted_iota(jnp.int32, (8, 128), 1)
    vals = jnp.zeros((8, 128), jnp.float32)
    for i, v in enumerate((s3a, s3b, s4a, s4b, s5a, s5b)):
        vals = jnp.where((row == 0) & (col == i), v, vals)
    out[...] = vals


def kernel(x_author, x_paper, edge_index_ap, edge_index_pa, W_author,
           b_author, W_paper, b_paper, Wq, bq, Wk, bk):
    N, D = x_author.shape
    H = W_author.shape[1]
    E = edge_index_ap.shape[1]

    qa, ka, qp, kp = _projections(x_author, x_paper, W_author, b_author,
                                  W_paper, b_paper, Wq, bq, Wk, bk)

    edge_kernel = _make_edge_kernel(N, H, E)
    ms = edge_kernel(qa, kp, qp, ka,
                     edge_index_ap[0], edge_index_ap[1],
                     edge_index_pa[0], edge_index_pa[1])

    res = pl.pallas_call(
        functools.partial(_combine_body, E=E),
        out_shape=jax.ShapeDtypeStruct((8, 128), jnp.float32),
    )(ms.reshape(4, _NW * _L))
    return res[0, :6]
